# Initial kernel scaffold; baseline (speedup 1.0000x reference)
import jax
import jax.numpy as jnp
from jax.experimental import pallas as pl


def kernel(x, train_data, train_labels):
    raise NotImplementedError("write your pallas kernel here")


# speedup vs baseline: 2.0575x; 2.0575x over previous
import functools

import jax
import jax.numpy as jnp
from jax.experimental import pallas as pl
from jax.experimental.pallas import tpu as pltpu

_NB = 2048
_K = 8
_C = 10
_I32MAX = 2**31 - 1


def _knn_body(x_ref, y_ref, x2_ref, y2_ref, lab_ref, pred_ref, proba_ref,
              bv_ref, bp_ref, *, nblk, n_true):
    i = pl.program_id(0)

    @pl.when(i == 0)
    def _init():
        bv_ref[...] = jnp.full(bv_ref.shape, jnp.inf, jnp.float32)
        bp_ref[...] = jnp.zeros(bp_ref.shape, jnp.int32)

    xq = x_ref[...]
    yb = y_ref[...]
    xy = jax.lax.dot_general(xq, yb, (((1,), (1,)), ((), ())),
                             preferred_element_type=jnp.float32)
    d2 = (x2_ref[...] + y2_ref[0]) - 2.0 * xy
    s = jnp.sqrt(jnp.maximum(d2, 0.0))
    colid = jax.lax.broadcasted_iota(jnp.int32, (1, _NB), 1) + i * _NB
    s = jnp.where(colid < n_true, s, jnp.inf)
    packed = colid * 16 + lab_ref[0]

    vals, pks = [], []
    scur = s
    for _ in range(_K):
        m = jnp.min(scur, axis=1, keepdims=True)
        pk = jnp.min(jnp.where(scur == m, packed, _I32MAX), axis=1,
                     keepdims=True)
        vals.append(m)
        pks.append(pk)
        scur = jnp.where(packed == pk, jnp.inf, scur)

    cv = jnp.concatenate([bv_ref[...]] + vals, axis=1)
    cp = jnp.concatenate([bp_ref[...]] + pks, axis=1)
    nvals, npks = [], []
    for _ in range(_K):
        m = jnp.min(cv, axis=1, keepdims=True)
        pk = jnp.min(jnp.where(cv == m, cp, _I32MAX), axis=1, keepdims=True)
        nvals.append(m)
        npks.append(pk)
        cv = jnp.where(cp == pk, jnp.inf, cv)
    bv = jnp.concatenate(nvals, axis=1)
    bp = jnp.concatenate(npks, axis=1)
    bv_ref[...] = bv
    bp_ref[...] = bp

    @pl.when(i == nblk - 1)
    def _finish():
        d = bv
        dinv = 1.0 / d
        inf_mask = jnp.isinf(dinv)
        inf_row = jnp.any(inf_mask, axis=1, keepdims=True)
        dinv = jnp.where(inf_row, inf_mask.astype(jnp.float32), dinv)
        lab = jnp.bitwise_and(bp, 15)
        cls = jax.lax.broadcasted_iota(jnp.int32, (1, _C), 1)
        proba = jnp.zeros((bv.shape[0], _C), jnp.float32)
        for k in range(_K):
            proba = proba + jnp.where(lab[:, k:k + 1] == cls,
                                      dinv[:, k:k + 1], 0.0)
        psum = jnp.sum(proba, axis=1, keepdims=True)
        psum = jnp.where(psum == 0.0, 1.0, psum)
        proba = proba * (1.0 / psum)
        pm = jnp.max(proba, axis=1, keepdims=True)
        pred = jnp.min(jnp.where(proba == pm, cls, _I32MAX), axis=1,
                       keepdims=True)
        pred_ref[...] = pred
        proba_ref[...] = proba


def kernel(x, train_data, train_labels):
    b, d = x.shape
    n = train_data.shape[0]
    nblk = (n + _NB - 1) // _NB
    npad = nblk * _NB
    td = jnp.pad(train_data, ((0, npad - n), (0, 0)))
    tl = jnp.pad(train_labels.astype(jnp.int32), (0, npad - n))
    tl = tl.reshape(nblk, 1, _NB)
    x2 = jnp.sum(x * x, axis=1, keepdims=True)
    y2 = jnp.pad(jnp.sum(train_data * train_data, axis=1), (0, npad - n))
    y23 = y2.reshape(nblk, 1, _NB)

    pred2, proba = pl.pallas_call(
        functools.partial(_knn_body, nblk=nblk, n_true=n),
        grid=(nblk,),
        in_specs=[
            pl.BlockSpec((b, d), lambda i: (0, 0)),
            pl.BlockSpec((_NB, d), lambda i: (i, 0)),
            pl.BlockSpec((b, 1), lambda i: (0, 0)),
            pl.BlockSpec((1, 1, _NB), lambda i: (i, 0, 0)),
            pl.BlockSpec((1, 1, _NB), lambda i: (i, 0, 0)),
        ],
        out_specs=[
            pl.BlockSpec((b, 1), lambda i: (0, 0)),
            pl.BlockSpec((b, _C), lambda i: (0, 0)),
        ],
        out_shape=[
            jax.ShapeDtypeStruct((b, 1), jnp.int32),
            jax.ShapeDtypeStruct((b, _C), jnp.float32),
        ],
        scratch_shapes=[
            pltpu.VMEM((b, _K), jnp.float32),
            pltpu.VMEM((b, _K), jnp.int32),
        ],
    )(x, td, x2, y23, tl)
    return pred2[:, 0], proba


# speedup vs baseline: 2.7835x; 1.3529x over previous
import functools

import jax
import jax.numpy as jnp
from jax import lax
from jax.experimental import pallas as pl
from jax.experimental.pallas import tpu as pltpu
from jax.experimental.pallas import tpu_sc as plsc

_NB = 2048
_G = 8
_NGB = _NB // _G
_K = 8
_C = 10
_QC = 128
_I32MAX = 2**31 - 1
_DIAG_XLA_GATHER = False
_DIAG_XLA_TOPK = False
_DIAG_XLA_K4 = False


def _k1_body(x_ref, y_ref, x2_ref, y2_ref, gm_ref, *, n_true):
    i = pl.program_id(0)
    xq = x_ref[...]
    yb = y_ref[...]
    xy = lax.dot_general(xq, yb, (((1,), (1,)), ((), ())),
                         preferred_element_type=jnp.float32)
    d2 = (x2_ref[...] + y2_ref[0]) - 2.0 * xy
    colid = lax.broadcasted_iota(jnp.int32, (1, _NB), 1) + i * _NB
    d2 = jnp.where(colid < n_true, d2, jnp.inf)
    g = jnp.min(d2.reshape(d2.shape[0], _G, _NGB), axis=1)
    gm_ref[...] = jnp.sqrt(jnp.maximum(g, 0.0))[None]


def _k2_body(gm_ref, cid_ref, *, ngrp):
    g = gm_ref[...]
    gid = lax.broadcasted_iota(jnp.int32, (1, ngrp), 1)
    pks = []
    for _ in range(_K):
        m = jnp.min(g, axis=1, keepdims=True)
        pk = jnp.min(jnp.where(g == m, gid, _I32MAX), axis=1, keepdims=True)
        pks.append(pk)
        g = jnp.where(gid == pk, jnp.inf, g)
    cid_ref[...] = jnp.concatenate(pks, axis=1)


def _k3_body(cids_hbm, aug_hbm, rows_out, cidbuf, idxbufa, idxbufb, rowsbufa,
             rowsbufb, sem1, *, num_cores):
    wid = lax.axis_index("s") * num_cores + lax.axis_index("c")
    lanes = lax.broadcasted_iota(jnp.int32, (16,), 0)
    for bt in range(8):
        qbase = (wid * 8 + bt) * 4
        for q in range(4):
            src = cids_hbm.at[pl.ds((qbase + q) * 8, 8)]
            pltpu.sync_copy(src, cidbuf.at[pl.ds(16 * q, 8)])
            pltpu.sync_copy(src, cidbuf.at[pl.ds(16 * q + 8, 8)])
        for c in range(16):
            q, m = c >> 2, c & 3
            cidrep = cidbuf[pl.ds(16 * q, 16)]
            jm = 2 * m + (lanes >> 3)
            idx = (2048 * (cidrep >> 8) + 256 * jm + (cidrep & 255))
            if c < 8:
                idxbufa[pl.ds(c * 16, 16)] = idx
            else:
                idxbufb[pl.ds((c - 8) * 16, 16)] = idx
        cpa = pltpu.async_copy(aug_hbm.at[idxbufa], rowsbufa, sem1)
        cpb = pltpu.async_copy(aug_hbm.at[idxbufb], rowsbufb, sem1)
        cpa.wait()
        cpb.wait()
        pltpu.sync_copy(rowsbufa, rows_out.at[pl.ds(qbase * 64, 128)])
        pltpu.sync_copy(rowsbufb, rows_out.at[pl.ds(qbase * 64 + 128, 128)])


def _k4_body(x_ref, x2_ref, rows_ref, y2c_ref, plab_ref, pred_ref, proba_ref,
             *, n_true):
    xq = x_ref[...]
    rows = rows_ref[0]
    xyf = lax.dot_general(xq, rows, (((1,), (1,)), ((), ())),
                          preferred_element_type=jnp.float32)
    colgrp = lax.broadcasted_iota(jnp.int32, (1, _QC * 64), 1) >> 6
    rowq = lax.broadcasted_iota(jnp.int32, (_QC, 1), 0)
    m = jnp.where(colgrp == rowq, xyf, 0.0)
    w = _QC * 64
    while w > 64:
        w //= 2
        m = m[:, :w] + m[:, w:2 * w]
    xyq = m

    d2 = (x2_ref[...] + y2c_ref[...]) - 2.0 * xyq
    key = jnp.sqrt(jnp.maximum(d2, 0.0))
    plabc = plab_ref[...]
    key = jnp.where((plabc >> 4) < n_true, key, jnp.inf)

    vals, pks = [], []
    for _ in range(_K):
        mn = jnp.min(key, axis=1, keepdims=True)
        pk = jnp.min(jnp.where(key == mn, plabc, _I32MAX), axis=1,
                     keepdims=True)
        vals.append(mn)
        pks.append(pk)
        key = jnp.where(plabc == pk, jnp.inf, key)
    d = jnp.concatenate(vals, axis=1)
    bp = jnp.concatenate(pks, axis=1)

    dinv = 1.0 / d
    inf_mask = jnp.isinf(dinv)
    inf_row = jnp.any(inf_mask, axis=1, keepdims=True)
    dinv = jnp.where(inf_row, inf_mask.astype(jnp.float32), dinv)
    labv = jnp.bitwise_and(bp, 15)
    cls = lax.broadcasted_iota(jnp.int32, (1, _C), 1)
    proba = jnp.zeros((_QC, _C), jnp.float32)
    for k in range(_K):
        proba = proba + jnp.where(labv[:, k:k + 1] == cls,
                                  dinv[:, k:k + 1], 0.0)
    psum = jnp.sum(proba, axis=1, keepdims=True)
    psum = jnp.where(psum == 0.0, 1.0, psum)
    proba = proba * (1.0 / psum)
    pm = jnp.max(proba, axis=1, keepdims=True)
    pred = jnp.min(jnp.where(proba == pm, cls, _I32MAX), axis=1,
                   keepdims=True)
    pred_ref[...] = pred
    proba_ref[...] = proba


def kernel(x, train_data, train_labels):
    b, dd = x.shape
    n = train_data.shape[0]
    nblk = (n + _NB - 1) // _NB
    npad = nblk * _NB
    ngrp = nblk * _NGB
    td = jnp.pad(train_data, ((0, npad - n), (0, 0)))
    tl = jnp.pad(train_labels.astype(jnp.int32), (0, npad - n))
    x2 = jnp.sum(x * x, axis=1, keepdims=True)
    y2 = jnp.pad(jnp.sum(train_data * train_data, axis=1), (0, npad - n))
    y23 = y2.reshape(nblk, 1, _NB)

    gm = pl.pallas_call(
        functools.partial(_k1_body, n_true=n),
        grid=(nblk,),
        in_specs=[
            pl.BlockSpec((b, dd), lambda i: (0, 0)),
            pl.BlockSpec((_NB, dd), lambda i: (i, 0)),
            pl.BlockSpec((b, 1), lambda i: (0, 0)),
            pl.BlockSpec((1, 1, _NB), lambda i: (i, 0, 0)),
        ],
        out_specs=pl.BlockSpec((1, b, _NGB), lambda i: (i, 0, 0)),
        out_shape=jax.ShapeDtypeStruct((nblk, b, _NGB), jnp.float32),
    )(x, td, x2, y23)
    gm = gm.transpose(1, 0, 2).reshape(b, ngrp)

    if _DIAG_XLA_TOPK:
        _, _tidx = jax.lax.top_k(-gm, _K)
        cids_x = _tidx.astype(jnp.int32)
    qb = 256
    cids = pl.pallas_call(
        functools.partial(_k2_body, ngrp=ngrp),
        grid=(b // qb,),
        in_specs=[pl.BlockSpec((qb, ngrp), lambda i: (i, 0))],
        out_specs=pl.BlockSpec((qb, _K), lambda i: (i, 0)),
        out_shape=jax.ShapeDtypeStruct((b, _K), jnp.int32),
    )(gm)
    if _DIAG_XLA_TOPK:
        cids = cids_x

    plab_full = jnp.arange(npad, dtype=jnp.int32) * 16 + tl
    aug = jnp.concatenate(
        [td, y2[:, None], plab_full.astype(jnp.float32)[:, None],
         jnp.zeros((npad, 128 - dd - 2), jnp.float32)], axis=1)

    rows_f = _sc_gather(cids.reshape(-1), aug, b)
    if _DIAG_XLA_GATHER:
        cids2 = cids.reshape(b, 1, 8)
        jj = jnp.arange(8).reshape(1, 8, 1)
        gidx = (2048 * (cids2 >> 8) + 256 * jj + (cids2 & 255)).reshape(-1)
        rows_f = aug[gidx]
    y2c_f = rows_f[:, 64].reshape(b, 64)
    plab_c = rows_f[:, 65].astype(jnp.int32).reshape(b, 64)
    xp = jnp.pad(x, ((0, 0), (0, 128 - dd)))

    if _DIAG_XLA_K4:
        rowsr = rows_f.reshape(b, 64, 128)
        xyq = jnp.einsum('qd,qcd->qc', xp, rowsr)
        d2c = (x2 + y2c_f) - 2.0 * xyq
        keyc = jnp.sqrt(jnp.maximum(d2c, 0.0))
        keyc = jnp.where((plab_c >> 4) < n, keyc, jnp.inf)
        negd, ci = jax.lax.top_k(-keyc, _K)
        dsel = -negd
        bp = jnp.take_along_axis(plab_c, ci, axis=1)
        dinv = jnp.power(dsel, -1.0)
        infm = jnp.isinf(dinv)
        infr = jnp.any(infm, axis=1)
        dinv = jnp.where(infr[:, None], infm.astype(jnp.float32), dinv)
        labv = jnp.bitwise_and(bp, 15)
        proba = jnp.zeros((b, _C), jnp.float32)
        proba = proba.at[jnp.arange(b)[:, None], labv].add(dinv)
        psum = jnp.sum(proba, axis=1, keepdims=True)
        psum = jnp.where(psum == 0, jnp.float32(1.0), psum)
        proba = jnp.power(psum, -1.0) * proba
        pred = jnp.argmax(proba, axis=1)
        return pred, proba

    nq = b // _QC
    pred2, proba = pl.pallas_call(
        functools.partial(_k4_body, n_true=n),
        grid=(nq,),
        in_specs=[
            pl.BlockSpec((_QC, 128), lambda i: (i, 0)),
            pl.BlockSpec((_QC, 1), lambda i: (i, 0)),
            pl.BlockSpec((1, _QC * 64, 128), lambda i: (i, 0, 0)),
            pl.BlockSpec((_QC, 64), lambda i: (i, 0)),
            pl.BlockSpec((_QC, 64), lambda i: (i, 0)),
        ],
        out_specs=[
            pl.BlockSpec((_QC, 1), lambda i: (i, 0)),
            pl.BlockSpec((_QC, _C), lambda i: (i, 0)),
        ],
        out_shape=[
            jax.ShapeDtypeStruct((b, 1), jnp.int32),
            jax.ShapeDtypeStruct((b, _C), jnp.float32),
        ],
    )(xp, x2, rows_f.reshape(nq, _QC * 64, 128), y2c_f, plab_c)
    return pred2[:, 0], proba


def _sc_gather(cids_flat, aug, b):
    info = plsc.get_sparse_core_info()
    gather = pl.kernel(
        functools.partial(_k3_body, num_cores=info.num_cores),
        out_type=jax.ShapeDtypeStruct((b * 64, 128), jnp.float32),
        mesh=plsc.VectorSubcoreMesh(core_axis_name="c", subcore_axis_name="s",
                                    num_cores=info.num_cores,
                                    num_subcores=info.num_subcores),
        scratch_types=[
            pltpu.VMEM((64,), jnp.int32),
            pltpu.VMEM((128,), jnp.int32),
            pltpu.VMEM((128,), jnp.int32),
            pltpu.VMEM((128, 128), jnp.float32),
            pltpu.VMEM((128, 128), jnp.float32),
            pltpu.SemaphoreType.DMA,
        ],
    )
    return gather(cids_flat, aug)


# speedup vs baseline: 4.6302x; 1.6635x over previous
import functools

import jax
import jax.numpy as jnp
from jax import lax
from jax.experimental import pallas as pl
from jax.experimental.pallas import tpu as pltpu
from jax.experimental.pallas import tpu_sc as plsc

_NB = 2048
_G = 8
_NGB = _NB // _G
_K = 8
_C = 10
_QC = 128
_I32MAX = 2**31 - 1
_DIAG_XLA_GATHER = False
_DIAG_XLA_TOPK = False
_DIAG_XLA_K4 = False


def _k1_body(x_ref, y_ref, x2_ref, y2_ref, gm_ref, *, n_true):
    i = pl.program_id(0)
    xq = x_ref[...]
    yb = y_ref[...]
    xy = lax.dot_general(xq, yb, (((1,), (1,)), ((), ())),
                         preferred_element_type=jnp.float32)
    d2 = (x2_ref[...] + y2_ref[0]) - 2.0 * xy
    colid = lax.broadcasted_iota(jnp.int32, (1, _NB), 1) + i * _NB
    d2 = jnp.where(colid < n_true, d2, jnp.inf)
    g = d2[:, 0:_NGB]
    for j in range(1, _G):
        g = jnp.minimum(g, d2[:, j * _NGB:(j + 1) * _NGB])
    gm_ref[...] = jnp.sqrt(jnp.maximum(g, 0.0))


def _k2_body(gm_ref, cid_ref, *, ngrp):
    g = gm_ref[...]
    gid = lax.broadcasted_iota(jnp.int32, (1, ngrp), 1)
    pks = []
    for _ in range(_K):
        m = jnp.min(g, axis=1, keepdims=True)
        pk = jnp.min(jnp.where(g == m, gid, _I32MAX), axis=1, keepdims=True)
        pks.append(pk)
        g = jnp.where(gid == pk, jnp.inf, g)
    cid_ref[...] = jnp.concatenate(pks, axis=1)


def _k3_body(cids_hbm, aug_hbm, rows_out, cidbuf, idxbufa, idxbufb, rowsbufa,
             rowsbufb, sem1, *, num_cores):
    wid = lax.axis_index("s") * num_cores + lax.axis_index("c")
    lanes = lax.broadcasted_iota(jnp.int32, (16,), 0)
    for bt in range(8):
        qbase = (wid * 8 + bt) * 4
        for q in range(4):
            src = cids_hbm.at[pl.ds((qbase + q) * 8, 8)]
            pltpu.sync_copy(src, cidbuf.at[pl.ds(16 * q, 8)])
            pltpu.sync_copy(src, cidbuf.at[pl.ds(16 * q + 8, 8)])
        for c in range(16):
            q, m = c >> 2, c & 3
            cidrep = cidbuf[pl.ds(16 * q, 16)]
            jm = 2 * m + (lanes >> 3)
            idx = (2048 * (cidrep >> 8) + 256 * jm + (cidrep & 255))
            if c < 8:
                idxbufa[pl.ds(c * 16, 16)] = idx
            else:
                idxbufb[pl.ds((c - 8) * 16, 16)] = idx
        cpa = pltpu.async_copy(aug_hbm.at[idxbufa], rowsbufa, sem1)
        cpb = pltpu.async_copy(aug_hbm.at[idxbufb], rowsbufb, sem1)
        cpa.wait()
        cpb.wait()
        pltpu.sync_copy(rowsbufa, rows_out.at[pl.ds(qbase * 64, 128)])
        pltpu.sync_copy(rowsbufb, rows_out.at[pl.ds(qbase * 64 + 128, 128)])


def _k4_body(x_ref, x2_ref, rows_ref, y2c_ref, plab_ref, pred_ref, proba_ref,
             *, n_true):
    xq = x_ref[...]
    rows = rows_ref[0]
    xyf = lax.dot_general(xq, rows, (((1,), (1,)), ((), ())),
                          preferred_element_type=jnp.float32)
    colgrp = lax.broadcasted_iota(jnp.int32, (1, _QC * 64), 1) >> 6
    rowq = lax.broadcasted_iota(jnp.int32, (_QC, 1), 0)
    m = jnp.where(colgrp == rowq, xyf, 0.0)
    w = _QC * 64
    while w > 64:
        w //= 2
        m = m[:, :w] + m[:, w:2 * w]
    xyq = m

    d2 = (x2_ref[...] + y2c_ref[...]) - 2.0 * xyq
    key = jnp.sqrt(jnp.maximum(d2, 0.0))
    plabc = plab_ref[...]
    key = jnp.where((plabc >> 4) < n_true, key, jnp.inf)

    vals, pks = [], []
    for _ in range(_K):
        mn = jnp.min(key, axis=1, keepdims=True)
        pk = jnp.min(jnp.where(key == mn, plabc, _I32MAX), axis=1,
                     keepdims=True)
        vals.append(mn)
        pks.append(pk)
        key = jnp.where(plabc == pk, jnp.inf, key)
    d = jnp.concatenate(vals, axis=1)
    bp = jnp.concatenate(pks, axis=1)

    dinv = 1.0 / d
    inf_mask = jnp.isinf(dinv)
    inf_row = jnp.any(inf_mask, axis=1, keepdims=True)
    dinv = jnp.where(inf_row, inf_mask.astype(jnp.float32), dinv)
    labv = jnp.bitwise_and(bp, 15)
    cls = lax.broadcasted_iota(jnp.int32, (1, _C), 1)
    proba = jnp.zeros((_QC, _C), jnp.float32)
    for k in range(_K):
        proba = proba + jnp.where(labv[:, k:k + 1] == cls,
                                  dinv[:, k:k + 1], 0.0)
    psum = jnp.sum(proba, axis=1, keepdims=True)
    psum = jnp.where(psum == 0.0, 1.0, psum)
    proba = proba * (1.0 / psum)
    pm = jnp.max(proba, axis=1, keepdims=True)
    pred = jnp.min(jnp.where(proba == pm, cls, _I32MAX), axis=1,
                   keepdims=True)
    pred_ref[...] = pred
    proba_ref[...] = proba


def kernel(x, train_data, train_labels):
    b, dd = x.shape
    n = train_data.shape[0]
    nblk = (n + _NB - 1) // _NB
    npad = nblk * _NB
    ngrp = nblk * _NGB
    td = jnp.pad(train_data, ((0, npad - n), (0, 0)))
    tl = jnp.pad(train_labels.astype(jnp.int32), (0, npad - n))
    x2 = jnp.sum(x * x, axis=1, keepdims=True)
    y2 = jnp.pad(jnp.sum(train_data * train_data, axis=1), (0, npad - n))
    y23 = y2.reshape(nblk, 1, _NB)

    gm = pl.pallas_call(
        functools.partial(_k1_body, n_true=n),
        grid=(nblk,),
        in_specs=[
            pl.BlockSpec((b, dd), lambda i: (0, 0)),
            pl.BlockSpec((_NB, dd), lambda i: (i, 0)),
            pl.BlockSpec((b, 1), lambda i: (0, 0)),
            pl.BlockSpec((1, 1, _NB), lambda i: (i, 0, 0)),
        ],
        out_specs=pl.BlockSpec((b, _NGB), lambda i: (0, i)),
        out_shape=jax.ShapeDtypeStruct((b, ngrp), jnp.float32),
    )(x, td, x2, y23)

    if _DIAG_XLA_TOPK:
        _, _tidx = jax.lax.top_k(-gm, _K)
        cids_x = _tidx.astype(jnp.int32)
    qb = 256
    cids = pl.pallas_call(
        functools.partial(_k2_body, ngrp=ngrp),
        grid=(b // qb,),
        in_specs=[pl.BlockSpec((qb, ngrp), lambda i: (i, 0))],
        out_specs=pl.BlockSpec((qb, _K), lambda i: (i, 0)),
        out_shape=jax.ShapeDtypeStruct((b, _K), jnp.int32),
    )(gm)
    if _DIAG_XLA_TOPK:
        cids = cids_x

    plab_full = jnp.arange(npad, dtype=jnp.int32) * 16 + tl
    aug = jnp.concatenate(
        [td, y2[:, None], plab_full.astype(jnp.float32)[:, None],
         jnp.zeros((npad, 128 - dd - 2), jnp.float32)], axis=1)

    rows_f = _sc_gather(cids.reshape(-1), aug, b)
    if _DIAG_XLA_GATHER:
        cids2 = cids.reshape(b, 1, 8)
        jj = jnp.arange(8).reshape(1, 8, 1)
        gidx = (2048 * (cids2 >> 8) + 256 * jj + (cids2 & 255)).reshape(-1)
        rows_f = aug[gidx]
    y2c_f = rows_f[:, 64].reshape(b, 64)
    plab_c = rows_f[:, 65].astype(jnp.int32).reshape(b, 64)
    xp = jnp.pad(x, ((0, 0), (0, 128 - dd)))

    if _DIAG_XLA_K4:
        rowsr = rows_f.reshape(b, 64, 128)
        xyq = jnp.einsum('qd,qcd->qc', xp, rowsr)
        d2c = (x2 + y2c_f) - 2.0 * xyq
        keyc = jnp.sqrt(jnp.maximum(d2c, 0.0))
        keyc = jnp.where((plab_c >> 4) < n, keyc, jnp.inf)
        negd, ci = jax.lax.top_k(-keyc, _K)
        dsel = -negd
        bp = jnp.take_along_axis(plab_c, ci, axis=1)
        dinv = jnp.power(dsel, -1.0)
        infm = jnp.isinf(dinv)
        infr = jnp.any(infm, axis=1)
        dinv = jnp.where(infr[:, None], infm.astype(jnp.float32), dinv)
        labv = jnp.bitwise_and(bp, 15)
        proba = jnp.zeros((b, _C), jnp.float32)
        proba = proba.at[jnp.arange(b)[:, None], labv].add(dinv)
        psum = jnp.sum(proba, axis=1, keepdims=True)
        psum = jnp.where(psum == 0, jnp.float32(1.0), psum)
        proba = jnp.power(psum, -1.0) * proba
        pred = jnp.argmax(proba, axis=1)
        return pred, proba

    nq = b // _QC
    pred2, proba = pl.pallas_call(
        functools.partial(_k4_body, n_true=n),
        grid=(nq,),
        in_specs=[
            pl.BlockSpec((_QC, 128), lambda i: (i, 0)),
            pl.BlockSpec((_QC, 1), lambda i: (i, 0)),
            pl.BlockSpec((1, _QC * 64, 128), lambda i: (i, 0, 0)),
            pl.BlockSpec((_QC, 64), lambda i: (i, 0)),
            pl.BlockSpec((_QC, 64), lambda i: (i, 0)),
        ],
        out_specs=[
            pl.BlockSpec((_QC, 1), lambda i: (i, 0)),
            pl.BlockSpec((_QC, _C), lambda i: (i, 0)),
        ],
        out_shape=[
            jax.ShapeDtypeStruct((b, 1), jnp.int32),
            jax.ShapeDtypeStruct((b, _C), jnp.float32),
        ],
    )(xp, x2, rows_f.reshape(nq, _QC * 64, 128), y2c_f, plab_c)
    return pred2[:, 0], proba


def _sc_gather(cids_flat, aug, b):
    info = plsc.get_sparse_core_info()
    gather = pl.kernel(
        functools.partial(_k3_body, num_cores=info.num_cores),
        out_type=jax.ShapeDtypeStruct((b * 64, 128), jnp.float32),
        mesh=plsc.VectorSubcoreMesh(core_axis_name="c", subcore_axis_name="s",
                                    num_cores=info.num_cores,
                                    num_subcores=info.num_subcores),
        scratch_types=[
            pltpu.VMEM((64,), jnp.int32),
            pltpu.VMEM((128,), jnp.int32),
            pltpu.VMEM((128,), jnp.int32),
            pltpu.VMEM((128, 128), jnp.float32),
            pltpu.VMEM((128, 128), jnp.float32),
            pltpu.SemaphoreType.DMA,
        ],
    )
    return gather(cids_flat, aug)


# speedup vs baseline: 4.7712x; 1.0304x over previous
import functools

import jax
import jax.numpy as jnp
from jax import lax
from jax.experimental import pallas as pl
from jax.experimental.pallas import tpu as pltpu
from jax.experimental.pallas import tpu_sc as plsc

_NB = 2048
_G = 8
_NGB = _NB // _G
_K = 8
_C = 10
_QC = 128
_I32MAX = 2**31 - 1
_DIAG_XLA_GATHER = False
_DIAG_XLA_TOPK = False
_DIAG_XLA_K4 = False


def _k1_body(x_ref, y_ref, x2_ref, y2_ref, gm_ref, *, n_true):
    del n_true
    xq = x_ref[...]
    yb = y_ref[...]
    xy = lax.dot_general(xq, yb, (((1,), (1,)), ((), ())),
                         preferred_element_type=jnp.float32)
    d2 = (x2_ref[...] + y2_ref[0]) - 2.0 * xy
    g = d2[:, 0:_NGB]
    for j in range(1, _G):
        g = jnp.minimum(g, d2[:, j * _NGB:(j + 1) * _NGB])
    gm_ref[...] = jnp.sqrt(jnp.maximum(g, 0.0))


def _k2_body(gm_ref, cid_ref, *, ngrp):
    g = gm_ref[...]
    gid = lax.broadcasted_iota(jnp.int32, (1, ngrp), 1)
    pks = []
    for _ in range(_K):
        m = jnp.min(g, axis=1, keepdims=True)
        pk = jnp.min(jnp.where(g == m, gid, _I32MAX), axis=1, keepdims=True)
        pks.append(pk)
        g = jnp.where(gid == pk, jnp.inf, g)
    cid_ref[...] = jnp.concatenate(pks, axis=1)


def _k3_body(cids_hbm, aug_hbm, rows_out, cidbuf, idxbufa, idxbufb, rowsbufa,
             rowsbufb, sem1, *, num_cores):
    wid = lax.axis_index("s") * num_cores + lax.axis_index("c")
    lanes = lax.broadcasted_iota(jnp.int32, (16,), 0)
    for bt in range(8):
        qbase = (wid * 8 + bt) * 4
        for q in range(4):
            src = cids_hbm.at[pl.ds((qbase + q) * 8, 8)]
            pltpu.sync_copy(src, cidbuf.at[pl.ds(16 * q, 8)])
            pltpu.sync_copy(src, cidbuf.at[pl.ds(16 * q + 8, 8)])
        for c in range(16):
            q, m = c >> 2, c & 3
            cidrep = cidbuf[pl.ds(16 * q, 16)]
            jm = 2 * m + (lanes >> 3)
            idx = (2048 * (cidrep >> 8) + 256 * jm + (cidrep & 255))
            if c < 8:
                idxbufa[pl.ds(c * 16, 16)] = idx
            else:
                idxbufb[pl.ds((c - 8) * 16, 16)] = idx
        cpa = pltpu.async_copy(aug_hbm.at[idxbufa], rowsbufa, sem1)
        cpb = pltpu.async_copy(aug_hbm.at[idxbufb], rowsbufb, sem1)
        cpa.wait()
        cpb.wait()
        pltpu.sync_copy(rowsbufa, rows_out.at[pl.ds(qbase * 64, 128)])
        pltpu.sync_copy(rowsbufb, rows_out.at[pl.ds(qbase * 64 + 128, 128)])


def _k4_body(x_ref, x2_ref, rows_ref, y2c_ref, plab_ref, pred_ref, proba_ref,
             *, n_true):
    xq = x_ref[...]
    rows = rows_ref[0]
    xyf = lax.dot_general(xq, rows, (((1,), (1,)), ((), ())),
                          preferred_element_type=jnp.float32)
    colgrp = lax.broadcasted_iota(jnp.int32, (1, _QC * 64), 1) >> 6
    rowq = lax.broadcasted_iota(jnp.int32, (_QC, 1), 0)
    m = jnp.where(colgrp == rowq, xyf, 0.0)
    w = _QC * 64
    while w > 64:
        w //= 2
        m = m[:, :w] + m[:, w:2 * w]
    xyq = m

    d2 = (x2_ref[...] + y2c_ref[...]) - 2.0 * xyq
    key = jnp.sqrt(jnp.maximum(d2, 0.0))
    plabc = plab_ref[...]
    key = jnp.where((plabc >> 4) < n_true, key, jnp.inf)

    vals, pks = [], []
    for _ in range(_K):
        mn = jnp.min(key, axis=1, keepdims=True)
        pk = jnp.min(jnp.where(key == mn, plabc, _I32MAX), axis=1,
                     keepdims=True)
        vals.append(mn)
        pks.append(pk)
        key = jnp.where(plabc == pk, jnp.inf, key)
    d = jnp.concatenate(vals, axis=1)
    bp = jnp.concatenate(pks, axis=1)

    dinv = 1.0 / d
    inf_mask = jnp.isinf(dinv)
    inf_row = jnp.any(inf_mask, axis=1, keepdims=True)
    dinv = jnp.where(inf_row, inf_mask.astype(jnp.float32), dinv)
    labv = jnp.bitwise_and(bp, 15)
    cls = lax.broadcasted_iota(jnp.int32, (1, _C), 1)
    proba = jnp.zeros((_QC, _C), jnp.float32)
    for k in range(_K):
        proba = proba + jnp.where(labv[:, k:k + 1] == cls,
                                  dinv[:, k:k + 1], 0.0)
    psum = jnp.sum(proba, axis=1, keepdims=True)
    psum = jnp.where(psum == 0.0, 1.0, psum)
    proba = proba * (1.0 / psum)
    pm = jnp.max(proba, axis=1, keepdims=True)
    pred = jnp.min(jnp.where(proba == pm, cls, _I32MAX), axis=1,
                   keepdims=True)
    pred_ref[...] = pred
    proba_ref[...] = proba


def kernel(x, train_data, train_labels):
    b, dd = x.shape
    n = train_data.shape[0]
    nblk = (n + _NB - 1) // _NB
    npad = nblk * _NB
    ngrp = nblk * _NGB
    td = jnp.pad(train_data, ((0, npad - n), (0, 0)))
    tl = jnp.pad(train_labels.astype(jnp.int32), (0, npad - n))
    x2 = jnp.sum(x * x, axis=1, keepdims=True)
    y2 = jnp.pad(jnp.sum(train_data * train_data, axis=1), (0, npad - n))
    y23 = jnp.pad(jnp.sum(train_data * train_data, axis=1), (0, npad - n),
                  constant_values=jnp.inf).reshape(nblk, 1, _NB)

    gm = pl.pallas_call(
        functools.partial(_k1_body, n_true=n),
        grid=(nblk,),
        in_specs=[
            pl.BlockSpec((b, dd), lambda i: (0, 0)),
            pl.BlockSpec((_NB, dd), lambda i: (i, 0)),
            pl.BlockSpec((b, 1), lambda i: (0, 0)),
            pl.BlockSpec((1, 1, _NB), lambda i: (i, 0, 0)),
        ],
        out_specs=pl.BlockSpec((b, _NGB), lambda i: (0, i)),
        out_shape=jax.ShapeDtypeStruct((b, ngrp), jnp.float32),
    )(x, td, x2, y23)

    if _DIAG_XLA_TOPK:
        _, _tidx = jax.lax.top_k(-gm, _K)
        cids_x = _tidx.astype(jnp.int32)
    qb = 256
    cids = pl.pallas_call(
        functools.partial(_k2_body, ngrp=ngrp),
        grid=(b // qb,),
        in_specs=[pl.BlockSpec((qb, ngrp), lambda i: (i, 0))],
        out_specs=pl.BlockSpec((qb, _K), lambda i: (i, 0)),
        out_shape=jax.ShapeDtypeStruct((b, _K), jnp.int32),
    )(gm)
    if _DIAG_XLA_TOPK:
        cids = cids_x

    plab_full = jnp.arange(npad, dtype=jnp.int32) * 16 + tl
    aug = jnp.concatenate(
        [td, y2[:, None], plab_full.astype(jnp.float32)[:, None],
         jnp.zeros((npad, 128 - dd - 2), jnp.float32)], axis=1)

    rows_f = _sc_gather(cids.reshape(-1), aug, b)
    if _DIAG_XLA_GATHER:
        cids2 = cids.reshape(b, 1, 8)
        jj = jnp.arange(8).reshape(1, 8, 1)
        gidx = (2048 * (cids2 >> 8) + 256 * jj + (cids2 & 255)).reshape(-1)
        rows_f = aug[gidx]
    y2c_f = rows_f[:, 64].reshape(b, 64)
    plab_c = rows_f[:, 65].astype(jnp.int32).reshape(b, 64)
    xp = jnp.pad(x, ((0, 0), (0, 128 - dd)))

    if _DIAG_XLA_K4:
        rowsr = rows_f.reshape(b, 64, 128)
        xyq = jnp.einsum('qd,qcd->qc', xp, rowsr)
        d2c = (x2 + y2c_f) - 2.0 * xyq
        keyc = jnp.sqrt(jnp.maximum(d2c, 0.0))
        keyc = jnp.where((plab_c >> 4) < n, keyc, jnp.inf)
        negd, ci = jax.lax.top_k(-keyc, _K)
        dsel = -negd
        bp = jnp.take_along_axis(plab_c, ci, axis=1)
        dinv = jnp.power(dsel, -1.0)
        infm = jnp.isinf(dinv)
        infr = jnp.any(infm, axis=1)
        dinv = jnp.where(infr[:, None], infm.astype(jnp.float32), dinv)
        labv = jnp.bitwise_and(bp, 15)
        proba = jnp.zeros((b, _C), jnp.float32)
        proba = proba.at[jnp.arange(b)[:, None], labv].add(dinv)
        psum = jnp.sum(proba, axis=1, keepdims=True)
        psum = jnp.where(psum == 0, jnp.float32(1.0), psum)
        proba = jnp.power(psum, -1.0) * proba
        pred = jnp.argmax(proba, axis=1)
        return pred, proba

    nq = b // _QC
    pred2, proba = pl.pallas_call(
        functools.partial(_k4_body, n_true=n),
        grid=(nq,),
        in_specs=[
            pl.BlockSpec((_QC, 128), lambda i: (i, 0)),
            pl.BlockSpec((_QC, 1), lambda i: (i, 0)),
            pl.BlockSpec((1, _QC * 64, 128), lambda i: (i, 0, 0)),
            pl.BlockSpec((_QC, 64), lambda i: (i, 0)),
            pl.BlockSpec((_QC, 64), lambda i: (i, 0)),
        ],
        out_specs=[
            pl.BlockSpec((_QC, 1), lambda i: (i, 0)),
            pl.BlockSpec((_QC, _C), lambda i: (i, 0)),
        ],
        out_shape=[
            jax.ShapeDtypeStruct((b, 1), jnp.int32),
            jax.ShapeDtypeStruct((b, _C), jnp.float32),
        ],
    )(xp, x2, rows_f.reshape(nq, _QC * 64, 128), y2c_f, plab_c)
    return pred2[:, 0], proba


def _sc_gather(cids_flat, aug, b):
    info = plsc.get_sparse_core_info()
    gather = pl.kernel(
        functools.partial(_k3_body, num_cores=info.num_cores),
        out_type=jax.ShapeDtypeStruct((b * 64, 128), jnp.float32),
        mesh=plsc.VectorSubcoreMesh(core_axis_name="c", subcore_axis_name="s",
                                    num_cores=info.num_cores,
                                    num_subcores=info.num_subcores),
        scratch_types=[
            pltpu.VMEM((64,), jnp.int32),
            pltpu.VMEM((128,), jnp.int32),
            pltpu.VMEM((128,), jnp.int32),
            pltpu.VMEM((128, 128), jnp.float32),
            pltpu.VMEM((128, 128), jnp.float32),
            pltpu.SemaphoreType.DMA,
        ],
    )
    return gather(cids_flat, aug)
